# baseline (speedup 1.0000x reference)
import functools

import jax
import jax.numpy as jnp
from jax import lax
from jax.experimental import pallas as pl
from jax.experimental.pallas import tpu as pltpu
from jax.experimental.pallas import tpu_sc as plsc

N = 10000
NPAD = 10112
E = 320000
CH = 128
HC = 64
OCH = 32
HW = 16
EC = 128
NCHUNK = E // EC
NW = 32
NSUB = 16
RPS = NPAD // NSUB
TMAX = 80
NB = 1264
G = NPAD // NB

_sc_mesh = plsc.VectorSubcoreMesh(core_axis_name="c", subcore_axis_name="s")


@functools.partial(
    pl.kernel,
    out_type=jax.ShapeDtypeStruct((2, NPAD, HW), jnp.float32),
    mesh=_sc_mesh,
    scratch_types=[
        pltpu.VMEM((2, EC), jnp.int32),
        pltpu.VMEM((EC, HW), jnp.float32),
        pltpu.VMEM((RPS, HW), jnp.float32),
        pltpu.VMEM_SHARED((NPAD, HW), jnp.float32),
        pltpu.SemaphoreType.DMA,
        pltpu.SemaphoreType.DMA,
    ],
    compiler_params=pltpu.CompilerParams(use_tc_tiling_on_sc=False),
)
def _hist_kernel(echunks, ones_hbm, zh_hbm, out, ibuf, ones_v, zb, hist_sh,
                 sem0, sem1):
    cid = lax.axis_index("c")
    sid = lax.axis_index("s")
    wid = cid * NSUB + sid
    sems = (sem0, sem1)

    pltpu.sync_copy(ones_hbm, ones_v)
    pltpu.sync_copy(zh_hbm, zb)
    pltpu.sync_copy(zb, hist_sh.at[pl.ds(sid * RPS, RPS)])
    plsc.subcore_barrier()

    def start(t, b):
        c = wid + t * NW

        @pl.when(c < NCHUNK)
        def _():
            pltpu.async_copy(echunks.at[c, 1], ibuf.at[b], sems[b])

    def finish(t, b):
        c = wid + t * NW

        @pl.when(c < NCHUNK)
        def _():
            pltpu.make_async_copy(echunks.at[c, 1], ibuf.at[b], sems[b]).wait()
            pltpu.sync_copy(ones_v, hist_sh.at[ibuf.at[b]], add=True)

    start(0, 0)

    def step2(i, carry):
        t = i * 2
        start(t + 1, 1)
        finish(t, 0)
        start(t + 2, 0)
        finish(t + 1, 1)
        return carry

    lax.fori_loop(0, TMAX // 2, step2, 0)

    plsc.subcore_barrier()
    pltpu.sync_copy(hist_sh.at[pl.ds(sid * RPS, RPS)], zb)
    pltpu.sync_copy(zb, out.at[cid, pl.ds(sid * RPS, RPS)])


@functools.partial(
    pl.kernel,
    out_type=(
        jax.ShapeDtypeStruct((2, NPAD, HC), jnp.float32),
        jax.ShapeDtypeStruct((2, NPAD, HC), jnp.float32),
    ),
    mesh=_sc_mesh,
    scratch_types=[
        pltpu.VMEM((TMAX, 2, EC), jnp.int32),
        pltpu.VMEM((2, EC, HC), jnp.float32),
        pltpu.VMEM((RPS, HC), jnp.float32),
        pltpu.VMEM_SHARED((NPAD, HC), jnp.float32),
        pltpu.SemaphoreType.DMA,
        pltpu.SemaphoreType.DMA,
        pltpu.SemaphoreType.DMA,
    ],
    compiler_params=pltpu.CompilerParams(use_tc_tiling_on_sc=False),
)
def _agg_kernel(hscA, hscB, echunks, outA, outB, ibuf, rows, zb, acc,
                isem, gs0, gs1):
    cid = lax.axis_index("c")
    sid = lax.axis_index("s")
    wid = cid * NSUB + sid
    gsems = (gs0, gs1)

    def pre_start(j):
        @pl.when(wid + j * NW < NCHUNK)
        def _():
            pltpu.async_copy(echunks.at[wid + j * NW], ibuf.at[j], isem)

    def pre_drain(j):
        @pl.when(wid + j * NW < NCHUNK)
        def _():
            pltpu.make_async_copy(echunks.at[wid + j * NW], ibuf.at[j],
                                  isem).wait()

    def pre_step(j, carry):
        pre_start(j)

        @pl.when(j >= 8)
        def _():
            pre_drain(j - 8)

        return carry

    lax.fori_loop(0, TMAX, pre_step, 0)

    def pre_tail(j, carry):
        pre_drain(j)
        return carry

    lax.fori_loop(TMAX - 8, TMAX, pre_tail, 0)

    zv = jnp.zeros((16,), jnp.float32)

    def zrow(r, carry):
        for l in range(HC // 16):
            zb[r, pl.ds(l * 16, 16)] = zv
        return carry

    lax.fori_loop(0, RPS, zrow, 0)

    for h, (hsrc, out) in enumerate(((hscA, outA), (hscB, outB))):

        def g_start(t, b, hsrc=hsrc):
            @pl.when(t * NW + wid < NCHUNK)
            def _():
                pltpu.async_copy(hsrc.at[ibuf.at[t, 0]], rows.at[b], gsems[b])

        def g_fin(t, b, hsrc=hsrc):
            @pl.when(t * NW + wid < NCHUNK)
            def _():
                pltpu.make_async_copy(hsrc.at[ibuf.at[t, 0]], rows.at[b],
                                      gsems[b]).wait()
                pltpu.sync_copy(rows.at[b], acc.at[ibuf.at[t, 1]], add=True)

        pltpu.sync_copy(zb, acc.at[pl.ds(sid * RPS, RPS)])
        plsc.subcore_barrier()

        g_start(0, 0)

        def step2(i, carry, g_start=g_start, g_fin=g_fin):
            t = i * 2
            g_start(t + 1, 1)
            g_fin(t, 0)
            g_start(t + 2, 0)
            g_fin(t + 1, 1)
            return carry

        lax.fori_loop(0, TMAX // 2, step2, 0)

        plsc.subcore_barrier()
        pltpu.sync_copy(acc.at[pl.ds(sid * RPS, RPS)], zb)
        pltpu.sync_copy(zb, out.at[cid, pl.ds(sid * RPS, RPS)])
        if h == 0:
            lax.fori_loop(0, RPS, zrow, 0)
        plsc.subcore_barrier()


def _tc0_body(x_ref, w0_ref, h_ref):
    h_ref[...] = jnp.dot(x_ref[...], w0_ref[...],
                         preferred_element_type=jnp.float32)


_tc0 = pl.pallas_call(
    _tc0_body,
    grid=(G,),
    in_specs=[
        pl.BlockSpec((NB, CH), lambda i: (i, 0)),
        pl.BlockSpec((CH, CH), lambda i: (0, 0)),
    ],
    out_specs=pl.BlockSpec((NB, CH), lambda i: (i, 0)),
    out_shape=jax.ShapeDtypeStruct((NPAD, CH), jnp.float32),
)


def _tc1_body(hist_ref, h_ref, dis_ref, hA_ref, hB_ref):
    hsum = hist_ref[0] + hist_ref[1]
    deg = hsum[:, 0:1] + 1.0
    dis = lax.rsqrt(deg)
    dis_ref[...] = dis
    h = h_ref[...] * dis
    hA_ref[...] = h[:, :HC]
    hB_ref[...] = h[:, HC:]


_tc1 = pl.pallas_call(
    _tc1_body,
    grid=(G,),
    in_specs=[
        pl.BlockSpec((2, NB, HW), lambda i: (0, i, 0)),
        pl.BlockSpec((NB, CH), lambda i: (i, 0)),
    ],
    out_specs=[
        pl.BlockSpec((NB, 1), lambda i: (i, 0)),
        pl.BlockSpec((NB, HC), lambda i: (i, 0)),
        pl.BlockSpec((NB, HC), lambda i: (i, 0)),
    ],
    out_shape=[
        jax.ShapeDtypeStruct((NPAD, 1), jnp.float32),
        jax.ShapeDtypeStruct((NPAD, HC), jnp.float32),
        jax.ShapeDtypeStruct((NPAD, HC), jnp.float32),
    ],
)


def _tc2_body(aggA_ref, aggB_ref, hA_ref, hB_ref, dis_ref, b_ref, w_ref,
              fl_ref, wc1_ref, bc1_ref, wc2_ref, bc2_ref,
              oA_ref, oB_ref, oc_ref):
    sA = aggA_ref[0] + aggA_ref[1] + hA_ref[...]
    sB = aggB_ref[0] + aggB_ref[1] + hB_ref[...]
    s = jnp.concatenate([sA, sB], axis=1)
    act = jnp.maximum(dis_ref[...] * s + b_ref[...], 0.0)
    h = jnp.dot(act, w_ref[...], preferred_element_type=jnp.float32)
    scale = jnp.where(fl_ref[...] > 0.0, dis_ref[...], 1.0)
    h = h * scale
    oA_ref[...] = h[:, :HC]
    oB_ref[...] = h[:, HC:]
    c = jnp.dot(act, wc1_ref[...], preferred_element_type=jnp.float32)
    c = jnp.maximum(c + bc1_ref[...], 0.0)
    logits = jnp.dot(c, wc2_ref[...], preferred_element_type=jnp.float32)
    oc_ref[...] = jax.nn.sigmoid(logits + bc2_ref[...])


_tc2 = pl.pallas_call(
    _tc2_body,
    grid=(G,),
    in_specs=[
        pl.BlockSpec((2, NB, HC), lambda i: (0, i, 0)),
        pl.BlockSpec((2, NB, HC), lambda i: (0, i, 0)),
        pl.BlockSpec((NB, HC), lambda i: (i, 0)),
        pl.BlockSpec((NB, HC), lambda i: (i, 0)),
        pl.BlockSpec((NB, 1), lambda i: (i, 0)),
        pl.BlockSpec((1, CH), lambda i: (0, 0)),
        pl.BlockSpec((CH, CH), lambda i: (0, 0)),
        pl.BlockSpec((1, 1), lambda i: (0, 0)),
        pl.BlockSpec((CH, CH), lambda i: (0, 0)),
        pl.BlockSpec((1, CH), lambda i: (0, 0)),
        pl.BlockSpec((CH, OCH), lambda i: (0, 0)),
        pl.BlockSpec((1, OCH), lambda i: (0, 0)),
    ],
    out_specs=[
        pl.BlockSpec((NB, HC), lambda i: (i, 0)),
        pl.BlockSpec((NB, HC), lambda i: (i, 0)),
        pl.BlockSpec((NB, OCH), lambda i: (i, 0)),
    ],
    out_shape=[
        jax.ShapeDtypeStruct((NPAD, HC), jnp.float32),
        jax.ShapeDtypeStruct((NPAD, HC), jnp.float32),
        jax.ShapeDtypeStruct((NPAD, OCH), jnp.float32),
    ],
)


def kernel(x, edge_index, batch, W0, b0, W1, b1, W2, b2, Wc1, bc1, Wc2, bc2):
    ei = edge_index.astype(jnp.int32)
    echunks = ei.reshape(2, NCHUNK, EC).transpose(1, 0, 2)
    ones_h = jnp.ones((EC, HW), jnp.float32)
    zh = jnp.zeros((RPS, HW), jnp.float32)
    xp = jnp.zeros((NPAD, CH), jnp.float32).at[:N].set(x)

    hist = _hist_kernel(echunks, ones_h, zh)
    h0 = _tc0(xp, W0)
    dis, hA, hB = _tc1(hist, h0)

    Wstack = jnp.stack([W1, W2, jnp.eye(CH, dtype=jnp.float32)])
    bstack = jnp.stack([b0.reshape(1, CH), b1.reshape(1, CH),
                        b2.reshape(1, CH)])
    flstack = jnp.array([[[1.0]], [[1.0]], [[-1.0]]], jnp.float32)
    bc1r = bc1.reshape(1, CH)
    bc2r = bc2.reshape(1, OCH)

    def layer(l, carry):
        chA, chB, _ = carry
        aggA, aggB = _agg_kernel(chA, chB, echunks)
        return _tc2(aggA, aggB, chA, chB, dis, bstack[l], Wstack[l],
                    flstack[l], Wc1, bc1r, Wc2, bc2r)

    _, _, oc = lax.fori_loop(
        0, 3, layer,
        (hA, hB, jnp.zeros((NPAD, OCH), jnp.float32)))
    return oc[:N]


# speedup vs baseline: 1.0114x; 1.0114x over previous
import functools

import jax
import jax.numpy as jnp
from jax import lax
from jax.experimental import pallas as pl
from jax.experimental.pallas import tpu as pltpu
from jax.experimental.pallas import tpu_sc as plsc

N = 10000
NPAD = 10112
E = 320000
CH = 128
HC = 64
OCH = 32
HW = 16
EC = 128
NCHUNK = E // EC
NW = 32
NSUB = 16
RPS = NPAD // NSUB
TMAX = 80
NB = 1264
G = NPAD // NB

_sc_mesh = plsc.VectorSubcoreMesh(core_axis_name="c", subcore_axis_name="s")


@functools.partial(
    pl.kernel,
    out_type=jax.ShapeDtypeStruct((2, NPAD, HW), jnp.float32),
    mesh=_sc_mesh,
    scratch_types=[
        pltpu.VMEM((2, EC), jnp.int32),
        pltpu.VMEM((EC, HW), jnp.float32),
        pltpu.VMEM((RPS, HW), jnp.float32),
        pltpu.VMEM_SHARED((NPAD, HW), jnp.float32),
        pltpu.SemaphoreType.DMA,
        pltpu.SemaphoreType.DMA,
    ],
    compiler_params=pltpu.CompilerParams(use_tc_tiling_on_sc=False),
)
def _hist_kernel(echunks, ones_hbm, zh_hbm, out, ibuf, ones_v, zb, hist_sh,
                 sem0, sem1):
    cid = lax.axis_index("c")
    sid = lax.axis_index("s")
    wid = cid * NSUB + sid
    sems = (sem0, sem1)

    pltpu.sync_copy(ones_hbm, ones_v)
    pltpu.sync_copy(zh_hbm, zb)
    pltpu.sync_copy(zb, hist_sh.at[pl.ds(sid * RPS, RPS)])
    plsc.subcore_barrier()

    def start(t, b):
        c = wid + t * NW

        @pl.when(c < NCHUNK)
        def _():
            pltpu.async_copy(echunks.at[c, 1], ibuf.at[b], sems[b])

    def finish(t, b):
        c = wid + t * NW

        @pl.when(c < NCHUNK)
        def _():
            pltpu.make_async_copy(echunks.at[c, 1], ibuf.at[b], sems[b]).wait()
            pltpu.sync_copy(ones_v, hist_sh.at[ibuf.at[b]], add=True)

    start(0, 0)

    def step2(i, carry):
        t = i * 2
        start(t + 1, 1)
        finish(t, 0)
        start(t + 2, 0)
        finish(t + 1, 1)
        return carry

    lax.fori_loop(0, TMAX // 2, step2, 0)

    plsc.subcore_barrier()
    pltpu.sync_copy(hist_sh.at[pl.ds(sid * RPS, RPS)], zb)
    pltpu.sync_copy(zb, out.at[cid, pl.ds(sid * RPS, RPS)])


@functools.partial(
    pl.kernel,
    out_type=(
        jax.ShapeDtypeStruct((2, NPAD, HC), jnp.float32),
        jax.ShapeDtypeStruct((2, NPAD, HC), jnp.float32),
    ),
    mesh=_sc_mesh,
    scratch_types=[
        pltpu.VMEM((TMAX, 2, EC), jnp.int32),
        pltpu.VMEM((2, EC, HC), jnp.float32),
        pltpu.VMEM((RPS, HC), jnp.float32),
        pltpu.VMEM_SHARED((NPAD, HC), jnp.float32),
        pltpu.SemaphoreType.DMA,
        pltpu.SemaphoreType.DMA,
        pltpu.SemaphoreType.DMA,
    ],
    compiler_params=pltpu.CompilerParams(use_tc_tiling_on_sc=False),
)
def _agg_kernel(hscA, hscB, echunks, outA, outB, ibuf, rows, zb, acc,
                isem, gs0, gs1):
    cid = lax.axis_index("c")
    sid = lax.axis_index("s")
    wid = cid * NSUB + sid
    gsems = (gs0, gs1)

    def pre_start(j):
        @pl.when(wid + j * NW < NCHUNK)
        def _():
            pltpu.async_copy(echunks.at[wid + j * NW], ibuf.at[j], isem)

    def pre_drain(j):
        @pl.when(wid + j * NW < NCHUNK)
        def _():
            pltpu.make_async_copy(echunks.at[wid + j * NW], ibuf.at[j],
                                  isem).wait()

    def pre_step(j, carry):
        pre_start(j)

        @pl.when(j >= 8)
        def _():
            pre_drain(j - 8)

        return carry

    lax.fori_loop(0, TMAX, pre_step, 0)

    def pre_tail(j, carry):
        pre_drain(j)
        return carry

    lax.fori_loop(TMAX - 8, TMAX, pre_tail, 0)

    zv = jnp.zeros((16,), jnp.float32)

    def zrow(r, carry):
        for l in range(HC // 16):
            zb[r, pl.ds(l * 16, 16)] = zv
        return carry

    lax.fori_loop(0, RPS, zrow, 0)

    for h, (hsrc, out) in enumerate(((hscA, outA), (hscB, outB))):

        def g_start(t, b, hsrc=hsrc):
            @pl.when(t * NW + wid < NCHUNK)
            def _():
                pltpu.async_copy(hsrc.at[ibuf.at[t, 0]], rows.at[b], gsems[b])

        def g_fin(t, b, hsrc=hsrc):
            @pl.when(t * NW + wid < NCHUNK)
            def _():
                pltpu.make_async_copy(hsrc.at[ibuf.at[t, 0]], rows.at[b],
                                      gsems[b]).wait()
                pltpu.sync_copy(rows.at[b], acc.at[ibuf.at[t, 1]], add=True)

        pltpu.sync_copy(zb, acc.at[pl.ds(sid * RPS, RPS)])
        plsc.subcore_barrier()

        g_start(0, 0)

        def step2(i, carry, g_start=g_start, g_fin=g_fin):
            t = i * 2
            g_start(t + 1, 1)
            g_fin(t, 0)
            g_start(t + 2, 0)
            g_fin(t + 1, 1)
            return carry

        lax.fori_loop(0, TMAX // 2, step2, 0)

        plsc.subcore_barrier()
        pltpu.sync_copy(acc.at[pl.ds(sid * RPS, RPS)], zb)
        pltpu.sync_copy(zb, out.at[cid, pl.ds(sid * RPS, RPS)])
        if h == 0:
            lax.fori_loop(0, RPS, zrow, 0)
        plsc.subcore_barrier()


def _tc1_body(hist_ref, x_ref, w0_ref, dis_ref, hA_ref, hB_ref):
    hsum = hist_ref[0] + hist_ref[1]
    deg = hsum[:, 0:1] + 1.0
    dis = lax.rsqrt(deg)
    dis_ref[...] = dis
    h = jnp.dot(x_ref[...], w0_ref[...], preferred_element_type=jnp.float32)
    h = h * dis
    hA_ref[...] = h[:, :HC]
    hB_ref[...] = h[:, HC:]


_tc1 = pl.pallas_call(
    _tc1_body,
    grid=(G,),
    in_specs=[
        pl.BlockSpec((2, NB, HW), lambda i: (0, i, 0)),
        pl.BlockSpec((NB, CH), lambda i: (i, 0)),
        pl.BlockSpec((CH, CH), lambda i: (0, 0)),
    ],
    out_specs=[
        pl.BlockSpec((NB, 1), lambda i: (i, 0)),
        pl.BlockSpec((NB, HC), lambda i: (i, 0)),
        pl.BlockSpec((NB, HC), lambda i: (i, 0)),
    ],
    out_shape=[
        jax.ShapeDtypeStruct((NPAD, 1), jnp.float32),
        jax.ShapeDtypeStruct((NPAD, HC), jnp.float32),
        jax.ShapeDtypeStruct((NPAD, HC), jnp.float32),
    ],
)


def _tc2_body(aggA_ref, aggB_ref, hA_ref, hB_ref, dis_ref, b_ref, w_ref,
              fl_ref, oA_ref, oB_ref):
    sA = aggA_ref[0] + aggA_ref[1] + hA_ref[...]
    sB = aggB_ref[0] + aggB_ref[1] + hB_ref[...]
    s = jnp.concatenate([sA, sB], axis=1)
    act = jnp.maximum(dis_ref[...] * s + b_ref[...], 0.0)
    h = jnp.dot(act, w_ref[...], preferred_element_type=jnp.float32)
    scale = jnp.where(fl_ref[...] > 0.0, dis_ref[...], 1.0)
    h = h * scale
    oA_ref[...] = h[:, :HC]
    oB_ref[...] = h[:, HC:]


_tc2 = pl.pallas_call(
    _tc2_body,
    grid=(G,),
    in_specs=[
        pl.BlockSpec((2, NB, HC), lambda i: (0, i, 0)),
        pl.BlockSpec((2, NB, HC), lambda i: (0, i, 0)),
        pl.BlockSpec((NB, HC), lambda i: (i, 0)),
        pl.BlockSpec((NB, HC), lambda i: (i, 0)),
        pl.BlockSpec((NB, 1), lambda i: (i, 0)),
        pl.BlockSpec((1, CH), lambda i: (0, 0)),
        pl.BlockSpec((CH, CH), lambda i: (0, 0)),
        pl.BlockSpec((1, 1), lambda i: (0, 0)),
    ],
    out_specs=[
        pl.BlockSpec((NB, HC), lambda i: (i, 0)),
        pl.BlockSpec((NB, HC), lambda i: (i, 0)),
    ],
    out_shape=[
        jax.ShapeDtypeStruct((NPAD, HC), jnp.float32),
        jax.ShapeDtypeStruct((NPAD, HC), jnp.float32),
    ],
)


def _tc4_body(aA_ref, aB_ref, wc1_ref, bc1_ref, wc2_ref, bc2_ref, out_ref):
    act = jnp.concatenate([aA_ref[...], aB_ref[...]], axis=1)
    c = jnp.dot(act, wc1_ref[...], preferred_element_type=jnp.float32)
    c = jnp.maximum(c + bc1_ref[...], 0.0)
    logits = jnp.dot(c, wc2_ref[...], preferred_element_type=jnp.float32)
    out_ref[...] = jax.nn.sigmoid(logits + bc2_ref[...])


_tc4 = pl.pallas_call(
    _tc4_body,
    grid=(G,),
    in_specs=[
        pl.BlockSpec((NB, HC), lambda i: (i, 0)),
        pl.BlockSpec((NB, HC), lambda i: (i, 0)),
        pl.BlockSpec((CH, CH), lambda i: (0, 0)),
        pl.BlockSpec((1, CH), lambda i: (0, 0)),
        pl.BlockSpec((CH, OCH), lambda i: (0, 0)),
        pl.BlockSpec((1, OCH), lambda i: (0, 0)),
    ],
    out_specs=pl.BlockSpec((NB, OCH), lambda i: (i, 0)),
    out_shape=jax.ShapeDtypeStruct((NPAD, OCH), jnp.float32),
)


def kernel(x, edge_index, batch, W0, b0, W1, b1, W2, b2, Wc1, bc1, Wc2, bc2):
    ei = edge_index.astype(jnp.int32)
    echunks = ei.reshape(2, NCHUNK, EC).transpose(1, 0, 2)
    ones_h = jnp.ones((EC, HW), jnp.float32)
    zh = jnp.zeros((RPS, HW), jnp.float32)
    xp = jnp.zeros((NPAD, CH), jnp.float32).at[:N].set(x)

    hist = _hist_kernel(echunks, ones_h, zh)
    dis, hA, hB = _tc1(hist, xp, W0)

    Wstack = jnp.stack([W1, W2, jnp.eye(CH, dtype=jnp.float32)])
    bstack = jnp.stack([b0.reshape(1, CH), b1.reshape(1, CH),
                        b2.reshape(1, CH)])
    flstack = jnp.array([[[1.0]], [[1.0]], [[-1.0]]], jnp.float32)

    def layer(l, carry):
        chA, chB = carry
        aggA, aggB = _agg_kernel(chA, chB, echunks)
        oA, oB = _tc2(aggA, aggB, chA, chB, dis, bstack[l], Wstack[l],
                      flstack[l])
        return (oA, oB)

    actA, actB = lax.fori_loop(0, 3, layer, (hA, hB))
    out = _tc4(actA, actB, Wc1, bc1.reshape(1, CH), Wc2,
               bc2.reshape(1, OCH))
    return out[:N]


# speedup vs baseline: 1.0712x; 1.0591x over previous
import functools

import jax
import jax.numpy as jnp
from jax import lax
from jax.experimental import pallas as pl
from jax.experimental.pallas import tpu as pltpu
from jax.experimental.pallas import tpu_sc as plsc

N = 10000
NPAD = 10112
E = 320000
CH = 128
HC = 64
OCH = 32
HW = 8
EC = 128
NCHUNK = E // EC
NW = 32
NSUB = 16
RPS = NPAD // NSUB
TMAX = 80
NB = 1264
G = NPAD // NB

_sc_mesh = plsc.VectorSubcoreMesh(core_axis_name="c", subcore_axis_name="s")


@functools.partial(
    pl.kernel,
    out_type=jax.ShapeDtypeStruct((2, NPAD, HW), jnp.float32),
    mesh=_sc_mesh,
    scratch_types=[
        pltpu.VMEM((2, EC), jnp.int32),
        pltpu.VMEM((EC, HW), jnp.float32),
        pltpu.VMEM((RPS, HW), jnp.float32),
        pltpu.VMEM_SHARED((NPAD, HW), jnp.float32),
        pltpu.SemaphoreType.DMA,
        pltpu.SemaphoreType.DMA,
    ],
    compiler_params=pltpu.CompilerParams(use_tc_tiling_on_sc=False),
)
def _hist_kernel(echunks, ones_hbm, zh_hbm, out, ibuf, ones_v, zb, hist_sh,
                 sem0, sem1):
    cid = lax.axis_index("c")
    sid = lax.axis_index("s")
    wid = cid * NSUB + sid
    sems = (sem0, sem1)

    pltpu.sync_copy(ones_hbm, ones_v)
    pltpu.sync_copy(zh_hbm, zb)
    pltpu.sync_copy(zb, hist_sh.at[pl.ds(sid * RPS, RPS)])
    plsc.subcore_barrier()

    def start(t, b):
        c = wid + t * NW

        @pl.when(c < NCHUNK)
        def _():
            pltpu.async_copy(echunks.at[c, 1], ibuf.at[b], sems[b])

    def finish(t, b):
        c = wid + t * NW

        @pl.when(c < NCHUNK)
        def _():
            pltpu.make_async_copy(echunks.at[c, 1], ibuf.at[b], sems[b]).wait()
            pltpu.sync_copy(ones_v, hist_sh.at[ibuf.at[b]], add=True)

    start(0, 0)

    def step2(i, carry):
        t = i * 2
        start(t + 1, 1)
        finish(t, 0)
        start(t + 2, 0)
        finish(t + 1, 1)
        return carry

    lax.fori_loop(0, TMAX // 2, step2, 0)

    plsc.subcore_barrier()
    pltpu.sync_copy(hist_sh.at[pl.ds(sid * RPS, RPS)], zb)
    pltpu.sync_copy(zb, out.at[cid, pl.ds(sid * RPS, RPS)])


@functools.partial(
    pl.kernel,
    out_type=(
        jax.ShapeDtypeStruct((2, NPAD, HC), jnp.float32),
        jax.ShapeDtypeStruct((2, NPAD, HC), jnp.float32),
    ),
    mesh=_sc_mesh,
    scratch_types=[
        pltpu.VMEM((TMAX, 2, EC), jnp.int32),
        pltpu.VMEM((4, EC, HC), jnp.float32),
        pltpu.VMEM_SHARED((NPAD, HC), jnp.float32),
        pltpu.SemaphoreType.DMA,
        pltpu.SemaphoreType.DMA,
        pltpu.SemaphoreType.DMA,
        pltpu.SemaphoreType.DMA,
        pltpu.SemaphoreType.DMA,
        pltpu.SemaphoreType.DMA,
        pltpu.SemaphoreType.DMA,
        pltpu.SemaphoreType.DMA,
        pltpu.SemaphoreType.DMA,
    ],
    compiler_params=pltpu.CompilerParams(use_tc_tiling_on_sc=False),
)
def _agg_kernel(hscA, hscB, echunks, zrows, outA, outB, ibuf, rows, acc,
                isem, gs0, gs1, gs2, gs3, ss0, ss1, ss2, ss3):
    cid = lax.axis_index("c")
    sid = lax.axis_index("s")
    wid = cid * NSUB + sid
    gsems = (gs0, gs1, gs2, gs3)
    ssems = (ss0, ss1, ss2, ss3)

    def pre_start(j):
        @pl.when(wid + j * NW < NCHUNK)
        def _():
            pltpu.async_copy(echunks.at[wid + j * NW], ibuf.at[j], isem)

    def pre_drain(j):
        @pl.when(wid + j * NW < NCHUNK)
        def _():
            pltpu.make_async_copy(echunks.at[wid + j * NW], ibuf.at[j],
                                  isem).wait()

    def pre_step(j, carry):
        pre_start(j)

        @pl.when(j >= 8)
        def _():
            pre_drain(j - 8)

        return carry

    lax.fori_loop(0, TMAX, pre_step, 0)

    def pre_tail(j, carry):
        pre_drain(j)
        return carry

    lax.fori_loop(TMAX - 8, TMAX, pre_tail, 0)

    for h, (hsrc, out) in enumerate(((hscA, outA), (hscB, outB))):

        def g_start(t, b, hsrc=hsrc):
            @pl.when((t >= 4) & ((t - 4) * NW + wid < NCHUNK))
            def _():
                pltpu.make_async_copy(rows.at[b], acc.at[ibuf.at[t - 4, 1]],
                                      ssems[b]).wait()

            @pl.when(t * NW + wid < NCHUNK)
            def _():
                pltpu.async_copy(hsrc.at[ibuf.at[t, 0]], rows.at[b], gsems[b])

        def g_fin(t, b, hsrc=hsrc):
            @pl.when(t * NW + wid < NCHUNK)
            def _():
                pltpu.make_async_copy(hsrc.at[ibuf.at[t, 0]], rows.at[b],
                                      gsems[b]).wait()
                pltpu.async_copy(rows.at[b], acc.at[ibuf.at[t, 1]], ssems[b],
                                 add=True)

        pltpu.sync_copy(zrows, acc.at[pl.ds(sid * RPS, RPS)])
        plsc.subcore_barrier()

        for b in range(4):
            g_start(b, b)

        def step4(i, carry, g_start=g_start, g_fin=g_fin):
            t = i * 4
            for b in range(4):
                g_fin(t + b, b)
            for b in range(4):
                g_start(t + 4 + b, b)
            return carry

        lax.fori_loop(0, TMAX // 4, step4, 0)

        plsc.subcore_barrier()
        pltpu.sync_copy(acc.at[pl.ds(sid * RPS, RPS)],
                        out.at[cid, pl.ds(sid * RPS, RPS)])
        plsc.subcore_barrier()


def _tc1_body(hist_ref, x_ref, w0_ref, dis_ref, hA_ref, hB_ref):
    hsum = hist_ref[0] + hist_ref[1]
    deg = hsum[:, 0:1] + 1.0
    dis = lax.rsqrt(deg)
    dis_ref[...] = dis
    h = jnp.dot(x_ref[...], w0_ref[...], preferred_element_type=jnp.float32)
    h = h * dis
    hA_ref[...] = h[:, :HC]
    hB_ref[...] = h[:, HC:]


_tc1 = pl.pallas_call(
    _tc1_body,
    grid=(G,),
    in_specs=[
        pl.BlockSpec((2, NB, HW), lambda i: (0, i, 0)),
        pl.BlockSpec((NB, CH), lambda i: (i, 0)),
        pl.BlockSpec((CH, CH), lambda i: (0, 0)),
    ],
    out_specs=[
        pl.BlockSpec((NB, 1), lambda i: (i, 0)),
        pl.BlockSpec((NB, HC), lambda i: (i, 0)),
        pl.BlockSpec((NB, HC), lambda i: (i, 0)),
    ],
    out_shape=[
        jax.ShapeDtypeStruct((NPAD, 1), jnp.float32),
        jax.ShapeDtypeStruct((NPAD, HC), jnp.float32),
        jax.ShapeDtypeStruct((NPAD, HC), jnp.float32),
    ],
)


def _tc2_body(aggA_ref, aggB_ref, hA_ref, hB_ref, dis_ref, b_ref, w_ref,
              fl_ref, oA_ref, oB_ref):
    sA = aggA_ref[0] + aggA_ref[1] + hA_ref[...]
    sB = aggB_ref[0] + aggB_ref[1] + hB_ref[...]
    s = jnp.concatenate([sA, sB], axis=1)
    act = jnp.maximum(dis_ref[...] * s + b_ref[...], 0.0)
    h = jnp.dot(act, w_ref[...], preferred_element_type=jnp.float32)
    scale = jnp.where(fl_ref[...] > 0.0, dis_ref[...], 1.0)
    h = h * scale
    oA_ref[...] = h[:, :HC]
    oB_ref[...] = h[:, HC:]


_tc2 = pl.pallas_call(
    _tc2_body,
    grid=(G,),
    in_specs=[
        pl.BlockSpec((2, NB, HC), lambda i: (0, i, 0)),
        pl.BlockSpec((2, NB, HC), lambda i: (0, i, 0)),
        pl.BlockSpec((NB, HC), lambda i: (i, 0)),
        pl.BlockSpec((NB, HC), lambda i: (i, 0)),
        pl.BlockSpec((NB, 1), lambda i: (i, 0)),
        pl.BlockSpec((1, CH), lambda i: (0, 0)),
        pl.BlockSpec((CH, CH), lambda i: (0, 0)),
        pl.BlockSpec((1, 1), lambda i: (0, 0)),
    ],
    out_specs=[
        pl.BlockSpec((NB, HC), lambda i: (i, 0)),
        pl.BlockSpec((NB, HC), lambda i: (i, 0)),
    ],
    out_shape=[
        jax.ShapeDtypeStruct((NPAD, HC), jnp.float32),
        jax.ShapeDtypeStruct((NPAD, HC), jnp.float32),
    ],
)


def _tc4_body(aA_ref, aB_ref, wc1_ref, bc1_ref, wc2_ref, bc2_ref, out_ref):
    act = jnp.concatenate([aA_ref[...], aB_ref[...]], axis=1)
    c = jnp.dot(act, wc1_ref[...], preferred_element_type=jnp.float32)
    c = jnp.maximum(c + bc1_ref[...], 0.0)
    logits = jnp.dot(c, wc2_ref[...], preferred_element_type=jnp.float32)
    out_ref[...] = jax.nn.sigmoid(logits + bc2_ref[...])


_tc4 = pl.pallas_call(
    _tc4_body,
    grid=(G,),
    in_specs=[
        pl.BlockSpec((NB, HC), lambda i: (i, 0)),
        pl.BlockSpec((NB, HC), lambda i: (i, 0)),
        pl.BlockSpec((CH, CH), lambda i: (0, 0)),
        pl.BlockSpec((1, CH), lambda i: (0, 0)),
        pl.BlockSpec((CH, OCH), lambda i: (0, 0)),
        pl.BlockSpec((1, OCH), lambda i: (0, 0)),
    ],
    out_specs=pl.BlockSpec((NB, OCH), lambda i: (i, 0)),
    out_shape=jax.ShapeDtypeStruct((NPAD, OCH), jnp.float32),
)


def kernel(x, edge_index, batch, W0, b0, W1, b1, W2, b2, Wc1, bc1, Wc2, bc2):
    ei = edge_index.astype(jnp.int32)
    echunks = ei.reshape(2, NCHUNK, EC).transpose(1, 0, 2)
    ones_h = jnp.ones((EC, HW), jnp.float32)
    zh = jnp.zeros((RPS, HW), jnp.float32)
    xp = jnp.zeros((NPAD, CH), jnp.float32).at[:N].set(x)

    hist = _hist_kernel(echunks, ones_h, zh)
    dis, hA, hB = _tc1(hist, xp, W0)

    Wstack = jnp.stack([W1, W2, jnp.eye(CH, dtype=jnp.float32)])
    bstack = jnp.stack([b0.reshape(1, CH), b1.reshape(1, CH),
                        b2.reshape(1, CH)])
    flstack = jnp.array([[[1.0]], [[1.0]], [[-1.0]]], jnp.float32)
    zrows = jnp.zeros((RPS, HC), jnp.float32)

    def layer(l, carry):
        chA, chB = carry
        aggA, aggB = _agg_kernel(chA, chB, echunks, zrows)
        oA, oB = _tc2(aggA, aggB, chA, chB, dis, bstack[l], Wstack[l],
                      flstack[l])
        return (oA, oB)

    actA, actB = lax.fori_loop(0, 3, layer, (hA, hB))
    out = _tc4(actA, actB, Wc1, bc1.reshape(1, CH), Wc2,
               bc2.reshape(1, OCH))
    return out[:N]
